# baseline (device time: 23900 ns/iter reference)
import jax
import jax.numpy as jnp
from jax import lax
from jax.experimental import pallas as pl
from jax.experimental.pallas import tpu as pltpu

N_DEV = 4
B, SQ, SKV, D_MODEL = 2, 256, 256, 512
H_LOC, DH = 4, 64
BLK = 64
QROWS = (B * SQ) // N_DEV


def kernel(x, Wq, K_ext, V_ext, Wo):
    def body(x_ref, wq_ref, k_hbm, v_hbm, wo_ref, out_ref,
             partial_ref, rs_recv, red_ref, k_scr, v_scr,
             rs_send_sems, rs_recv_sems, ag_send_sems, ag_recv_sems,
             kv_sems):
        my = lax.axis_index("i")

        h0 = my * H_LOC
        cp_k = pltpu.make_async_copy(
            k_hbm.at[:, :, pl.ds(h0, H_LOC), :], k_scr, kv_sems.at[0])
        cp_v = pltpu.make_async_copy(
            v_hbm.at[:, :, pl.ds(h0, H_LOC), :], v_scr, kv_sems.at[1])
        cp_k.start()
        cp_v.start()

        barrier = pltpu.get_barrier_semaphore()
        for j in range(1, N_DEV):
            pl.semaphore_signal(barrier, inc=1,
                                device_id=(lax.rem(my + j, N_DEV),),
                                device_id_type=pl.DeviceIdType.MESH)
        pl.semaphore_wait(barrier, N_DEV - 1)

        wq = wq_ref[...].astype(jnp.bfloat16)
        wo = wo_ref[...].astype(jnp.bfloat16)
        cols_blk = lax.broadcasted_iota(jnp.int32, (QROWS, SKV), 1) // BLK
        rows_iota = lax.broadcasted_iota(jnp.int32, (QROWS, SKV), 0)
        cp_k.wait()
        cp_v.wait()

        def compute_quarter(qtr):
            b = qtr // 2
            r0 = (qtr % 2) * QROWS
            xb = x_ref[b, pl.ds(r0, QROWS), :].astype(jnp.bfloat16)
            qp = jnp.dot(xb, wq, preferred_element_type=jnp.float32)
            mask = cols_blk <= (r0 + rows_iota) // BLK
            ctx = []
            for h in range(H_LOC):
                qh = qp[:, h * DH:(h + 1) * DH].astype(jnp.bfloat16)
                kh = k_scr[b, :, h, :].astype(jnp.bfloat16)
                vh = v_scr[b, :, h, :].astype(jnp.bfloat16)
                s = lax.dot_general(qh, kh, (((1,), (1,)), ((), ())),
                                    preferred_element_type=jnp.float32) * 0.125
                w = jnp.exp(jnp.where(mask, s, -1e9))
                denom = jnp.sum(w, axis=-1, keepdims=True)
                pv = jnp.dot(w.astype(jnp.bfloat16), vh,
                             preferred_element_type=jnp.float32)
                ctx.append(pv / denom)
            ctx_all = jnp.concatenate(ctx, axis=1).astype(jnp.bfloat16)
            return jnp.dot(ctx_all, wo, preferred_element_type=jnp.float32)

        rs_rdmas = []
        for j in range(1, N_DEV):
            qtr = lax.rem(my + j, N_DEV)
            partial_ref[j - 1] = compute_quarter(qtr).astype(jnp.bfloat16)
            rdma = pltpu.make_async_remote_copy(
                src_ref=partial_ref.at[j - 1],
                dst_ref=rs_recv.at[N_DEV - j],
                send_sem=rs_send_sems.at[j - 1],
                recv_sem=rs_recv_sems.at[N_DEV - j],
                device_id=(qtr,),
                device_id_type=pl.DeviceIdType.MESH,
            )
            rdma.start()
            rs_rdmas.append(rdma)

        own = compute_quarter(my)

        for k in range(1, N_DEV):
            pltpu.make_async_remote_copy(
                src_ref=rs_recv.at[k], dst_ref=rs_recv.at[k],
                send_sem=rs_send_sems.at[0], recv_sem=rs_recv_sems.at[k],
                device_id=(my,), device_id_type=pl.DeviceIdType.MESH,
            ).wait_recv()

        acc = (own + rs_recv[1].astype(jnp.float32)
               + rs_recv[2].astype(jnp.float32)
               + rs_recv[3].astype(jnp.float32))
        red = acc.astype(jnp.bfloat16)
        red_ref[...] = red
        my_b = my // 2
        my_row = (my % 2) * QROWS
        out_ref[my_b, pl.ds(my_row, QROWS), :] = red

        ag_rdmas = []
        for j in range(1, N_DEV):
            rdma = pltpu.make_async_remote_copy(
                src_ref=red_ref,
                dst_ref=out_ref.at[my_b, pl.ds(my_row, QROWS), :],
                send_sem=ag_send_sems.at[j - 1],
                recv_sem=ag_recv_sems.at[N_DEV - j],
                device_id=(lax.rem(my + j, N_DEV),),
                device_id_type=pl.DeviceIdType.MESH,
            )
            rdma.start()
            ag_rdmas.append(rdma)

        for k in range(1, N_DEV):
            pltpu.make_async_remote_copy(
                src_ref=red_ref,
                dst_ref=out_ref.at[0, pl.ds(0, QROWS), :],
                send_sem=ag_send_sems.at[0], recv_sem=ag_recv_sems.at[k],
                device_id=(my,), device_id_type=pl.DeviceIdType.MESH,
            ).wait_recv()

        for rdma in rs_rdmas + ag_rdmas:
            rdma.wait_send()

    return pl.pallas_call(
        body,
        out_shape=jax.ShapeDtypeStruct((B, SQ, D_MODEL), jnp.bfloat16),
        in_specs=[
            pl.BlockSpec(memory_space=pltpu.VMEM),
            pl.BlockSpec(memory_space=pltpu.VMEM),
            pl.BlockSpec(memory_space=pl.ANY),
            pl.BlockSpec(memory_space=pl.ANY),
            pl.BlockSpec(memory_space=pltpu.VMEM),
        ],
        out_specs=pl.BlockSpec(memory_space=pltpu.VMEM),
        scratch_shapes=[
            pltpu.VMEM((N_DEV - 1, QROWS, D_MODEL), jnp.bfloat16),
            pltpu.VMEM((N_DEV, QROWS, D_MODEL), jnp.bfloat16),
            pltpu.VMEM((QROWS, D_MODEL), jnp.bfloat16),
            pltpu.VMEM((B, SKV, H_LOC, DH), jnp.float32),
            pltpu.VMEM((B, SKV, H_LOC, DH), jnp.float32),
            pltpu.SemaphoreType.DMA((N_DEV - 1,)),
            pltpu.SemaphoreType.DMA((N_DEV,)),
            pltpu.SemaphoreType.DMA((N_DEV - 1,)),
            pltpu.SemaphoreType.DMA((N_DEV,)),
            pltpu.SemaphoreType.DMA((2,)),
        ],
        compiler_params=pltpu.CompilerParams(collective_id=0),
    )(x, Wq, K_ext, V_ext, Wo)


# device time: 16902 ns/iter; 1.4140x vs baseline; 1.4140x over previous
import jax
import jax.numpy as jnp
from jax import lax
from jax.experimental import pallas as pl
from jax.experimental.pallas import tpu as pltpu

N_DEV = 4
B, SQ, SKV, D_MODEL = 2, 256, 256, 512
H_LOC, DH = 4, 64
BLK = 64
QROWS = (B * SQ) // N_DEV


def kernel(x, Wq, K_ext, V_ext, Wo):
    i = lax.axis_index("i")
    K = lax.dynamic_slice_in_dim(K_ext, i * H_LOC, H_LOC, axis=2).reshape(
        B, SKV, H_LOC * DH)
    V = lax.dynamic_slice_in_dim(V_ext, i * H_LOC, H_LOC, axis=2).reshape(
        B, SKV, H_LOC * DH)

    def body(x_ref, wq_ref, k_scr, v_scr, wo_ref, out_ref,
             partial_ref, rs_recv, red_ref,
             rs_send_sems, rs_recv_sems, ag_send_sems, ag_recv_sems):
        my = lax.axis_index("i")

        barrier = pltpu.get_barrier_semaphore()
        for j in range(1, N_DEV):
            pl.semaphore_signal(barrier, inc=1,
                                device_id=(lax.rem(my + j, N_DEV),),
                                device_id_type=pl.DeviceIdType.MESH)
        pl.semaphore_wait(barrier, N_DEV - 1)

        wq = wq_ref[...].astype(jnp.bfloat16)
        wo = wo_ref[...].astype(jnp.bfloat16)
        cols_blk = lax.broadcasted_iota(jnp.int32, (QROWS, SKV), 1) // BLK
        rows_iota = lax.broadcasted_iota(jnp.int32, (QROWS, SKV), 0)

        def compute_quarter(qtr):
            b = qtr // 2
            r0 = (qtr % 2) * QROWS
            xb = x_ref[b, pl.ds(r0, QROWS), :].astype(jnp.bfloat16)
            qp = jnp.dot(xb, wq, preferred_element_type=jnp.float32)
            mask = cols_blk <= (r0 + rows_iota) // BLK
            ctx = []
            for h in range(H_LOC):
                qh = qp[:, h * DH:(h + 1) * DH].astype(jnp.bfloat16)
                kh = k_scr[b, :, h * DH:(h + 1) * DH].astype(jnp.bfloat16)
                vh = v_scr[b, :, h * DH:(h + 1) * DH].astype(jnp.bfloat16)
                s = lax.dot_general(qh, kh, (((1,), (1,)), ((), ())),
                                    preferred_element_type=jnp.float32) * 0.125
                w = jnp.exp(jnp.where(mask, s, -1e9))
                denom = jnp.sum(w, axis=-1, keepdims=True)
                pv = jnp.dot(w.astype(jnp.bfloat16), vh,
                             preferred_element_type=jnp.float32)
                ctx.append(pv / denom)
            ctx_all = jnp.concatenate(ctx, axis=1).astype(jnp.bfloat16)
            return jnp.dot(ctx_all, wo, preferred_element_type=jnp.float32)

        rs_rdmas = []
        for j in range(1, N_DEV):
            qtr = lax.rem(my + j, N_DEV)
            partial_ref[j - 1] = compute_quarter(qtr).astype(jnp.bfloat16)
            rdma = pltpu.make_async_remote_copy(
                src_ref=partial_ref.at[j - 1],
                dst_ref=rs_recv.at[N_DEV - j],
                send_sem=rs_send_sems.at[j - 1],
                recv_sem=rs_recv_sems.at[N_DEV - j],
                device_id=(qtr,),
                device_id_type=pl.DeviceIdType.MESH,
            )
            rdma.start()
            rs_rdmas.append(rdma)

        own = compute_quarter(my)

        for k in range(1, N_DEV):
            pltpu.make_async_remote_copy(
                src_ref=rs_recv.at[k], dst_ref=rs_recv.at[k],
                send_sem=rs_send_sems.at[0], recv_sem=rs_recv_sems.at[k],
                device_id=(my,), device_id_type=pl.DeviceIdType.MESH,
            ).wait_recv()

        acc = (own + rs_recv[1].astype(jnp.float32)
               + rs_recv[2].astype(jnp.float32)
               + rs_recv[3].astype(jnp.float32))
        red = acc.astype(jnp.bfloat16)
        red_ref[...] = red
        my_b = my // 2
        my_row = (my % 2) * QROWS
        out_ref[my_b, pl.ds(my_row, QROWS), :] = red

        ag_rdmas = []
        for j in range(1, N_DEV):
            rdma = pltpu.make_async_remote_copy(
                src_ref=red_ref,
                dst_ref=out_ref.at[my_b, pl.ds(my_row, QROWS), :],
                send_sem=ag_send_sems.at[j - 1],
                recv_sem=ag_recv_sems.at[N_DEV - j],
                device_id=(lax.rem(my + j, N_DEV),),
                device_id_type=pl.DeviceIdType.MESH,
            )
            rdma.start()
            ag_rdmas.append(rdma)

        for k in range(1, N_DEV):
            pltpu.make_async_remote_copy(
                src_ref=red_ref,
                dst_ref=out_ref.at[0, pl.ds(0, QROWS), :],
                send_sem=ag_send_sems.at[0], recv_sem=ag_recv_sems.at[k],
                device_id=(my,), device_id_type=pl.DeviceIdType.MESH,
            ).wait_recv()

        for rdma in rs_rdmas + ag_rdmas:
            rdma.wait_send()

    return pl.pallas_call(
        body,
        out_shape=jax.ShapeDtypeStruct((B, SQ, D_MODEL), jnp.bfloat16),
        in_specs=[
            pl.BlockSpec(memory_space=pltpu.VMEM),
            pl.BlockSpec(memory_space=pltpu.VMEM),
            pl.BlockSpec(memory_space=pltpu.VMEM),
            pl.BlockSpec(memory_space=pltpu.VMEM),
            pl.BlockSpec(memory_space=pltpu.VMEM),
        ],
        out_specs=pl.BlockSpec(memory_space=pltpu.VMEM),
        scratch_shapes=[
            pltpu.VMEM((N_DEV - 1, QROWS, D_MODEL), jnp.bfloat16),
            pltpu.VMEM((N_DEV, QROWS, D_MODEL), jnp.bfloat16),
            pltpu.VMEM((QROWS, D_MODEL), jnp.bfloat16),
            pltpu.SemaphoreType.DMA((N_DEV - 1,)),
            pltpu.SemaphoreType.DMA((N_DEV,)),
            pltpu.SemaphoreType.DMA((N_DEV - 1,)),
            pltpu.SemaphoreType.DMA((N_DEV,)),
        ],
        compiler_params=pltpu.CompilerParams(collective_id=0),
    )(x, Wq, K, V, Wo)
